# baseline (speedup 1.0000x reference)
import jax
import jax.numpy as jnp
from jax import lax
from jax.experimental import pallas as pl
from jax.experimental.pallas import tpu as pltpu
from jax.experimental.pallas import tpu_sc as plsc

_NC = 2
_NS = 16
_NW = _NC * _NS

_ROWS = 1000000
_DIM = 32
_RPW = (_ROWS // _NW) // 32 * 32
_TAIL_BASE = _NW * _RPW
_TAIL_ROWS = _ROWS - _TAIL_BASE
_CHUNK = 256
_N_CHUNKS = _RPW // _CHUNK
_NBUF = 2
_TCHUNK = 192


def _sc_copy(in_hbm, out_hbm, buf, load_sems, store_sems):
    wid = lax.axis_index("s") * _NC + lax.axis_index("c")
    base = pl.multiple_of(wid * _RPW, 32)

    def body(k, carry):
        s = lax.rem(k, _NBUF)
        off = base + k * _CHUNK

        @pl.when(k >= _NBUF)
        def _():
            pltpu.make_async_copy(
                buf.at[s],
                out_hbm.at[pl.ds(0, _CHUNK)],
                store_sems.at[s],
            ).wait()

        lc = pltpu.make_async_copy(
            in_hbm.at[pl.ds(off, _CHUNK)], buf.at[s], load_sems.at[s]
        )
        lc.start()
        lc.wait()
        pltpu.make_async_copy(
            buf.at[s], out_hbm.at[pl.ds(off, _CHUNK)], store_sems.at[s]
        ).start()
        return carry

    lax.fori_loop(0, _N_CHUNKS, body, 0)
    for s in range(_NBUF):
        pltpu.make_async_copy(
            buf.at[s], out_hbm.at[pl.ds(0, _CHUNK)], store_sems.at[s]
        ).wait()

    @pl.when(wid == 0)
    def _():
        for i in range(_TAIL_ROWS // _TCHUNK):
            tb = buf.at[0].at[pl.ds(0, _TCHUNK)]
            off = _TAIL_BASE + i * _TCHUNK
            pltpu.sync_copy(in_hbm.at[pl.ds(off, _TCHUNK)], tb)
            pltpu.sync_copy(tb, out_hbm.at[pl.ds(off, _TCHUNK)])


def kernel(embeddings):
    mesh = plsc.VectorSubcoreMesh(core_axis_name="c", subcore_axis_name="s")
    run = pl.kernel(
        _sc_copy,
        out_type=jax.ShapeDtypeStruct(embeddings.shape, embeddings.dtype),
        mesh=mesh,
        scratch_types=[
            pltpu.VMEM((_NBUF, _CHUNK, _DIM), jnp.float32),
            pltpu.SemaphoreType.DMA((_NBUF,)),
            pltpu.SemaphoreType.DMA((_NBUF,)),
        ],
        compiler_params=pltpu.CompilerParams(skip_device_barrier=True),
    )
    return run(embeddings)
